# Initial kernel scaffold; baseline (speedup 1.0000x reference)
import jax
import jax.numpy as jnp
from jax.experimental import pallas as pl


def kernel(dist_mat, tree_embeds, tree_mask, a1, p, a2, n):
    raise NotImplementedError("write your pallas kernel here")


# speedup vs baseline: 3.2991x; 3.2991x over previous
import functools

import jax
import jax.numpy as jnp
from jax import lax
from jax.experimental import pallas as pl
from jax.experimental.pallas import tpu as pltpu
from jax.experimental.pallas import tpu_sc as plsc

_B = 4096
_P = 100000
_NC = 2
_NS = 16
_CH = 6272
_PPAD = _NS * _CH
_NCHK = _CH // 128
_SLOT_H = _B * _B + 8

_mesh = plsc.VectorSubcoreMesh(core_axis_name="c", subcore_axis_name="s")


def _build_keys(aa_hbm, yy_hbm, av, yv, keys, dkeys, tvals, cid, sid):
    base = cid * _PPAD + sid * _CH
    tbase = sid * _CH
    slot_off = cid * _SLOT_H

    pltpu.sync_copy(aa_hbm.at[pl.ds(base, _CH)], av)
    pltpu.sync_copy(yy_hbm.at[pl.ds(base, _CH)], yv)

    def build(c, carry):
        for j in range(8):
            o = c * 128 + j * 16
            a = av[pl.ds(o, 16)]
            y = yv[pl.ds(o, 16)]
            t = tbase + o + lax.iota(jnp.int32, 16)
            valid = t < _P
            k = a * _B + y
            keys[c, pl.ds(j * 16, 16)] = jnp.where(valid, k + slot_off,
                                                   slot_off + _B * _B)
            if dkeys is not None:
                dkeys[c, pl.ds(j * 16, 16)] = jnp.where(valid, k, 0)
            tvals[c, pl.ds(j * 16, 16)] = t
        return carry

    lax.fori_loop(0, _NCHK, build, 0)


@functools.partial(
    pl.kernel,
    mesh=_mesh,
    out_type=[
        jax.ShapeDtypeStruct((_NC * _SLOT_H,), jnp.int32),
    ],
    scratch_types=[
        pltpu.VMEM((_CH,), jnp.int32),
        pltpu.VMEM((_CH,), jnp.int32),
        pltpu.VMEM((_NCHK, 128), jnp.int32),
        pltpu.VMEM((_NCHK, 128), jnp.int32),
        pltpu.SemaphoreType.DMA,
    ],
)
def _sc_scatter(aa_hbm, yy_hbm, slot_out, av, yv, keys, tvals, sem_s):
    cid = lax.axis_index("c")
    sid = lax.axis_index("s")
    _build_keys(aa_hbm, yy_hbm, av, yv, keys, None, tvals, cid, sid)

    def fire_s(c, carry):
        pltpu.make_async_copy(tvals.at[c], slot_out.at[keys.at[c]], sem_s).start()
        return carry

    lax.fori_loop(0, _NCHK, fire_s, 0)

    def drain_s(c, carry):
        pltpu.make_async_copy(tvals.at[c], slot_out.at[keys.at[c]], sem_s).wait()
        return carry

    lax.fori_loop(0, _NCHK, drain_s, 0)


@functools.partial(
    pl.kernel,
    mesh=_mesh,
    out_type=[
        jax.ShapeDtypeStruct((_NC * _NS, _NCHK, 128), jnp.float32),
    ],
    scratch_types=[
        pltpu.VMEM((_CH,), jnp.int32),
        pltpu.VMEM((_CH,), jnp.int32),
        pltpu.VMEM((_NCHK, 128), jnp.int32),
        pltpu.VMEM((_NCHK, 128), jnp.int32),
        pltpu.VMEM((_NCHK, 128), jnp.int32),
        pltpu.VMEM((_NCHK, 128), jnp.int32),
        pltpu.VMEM((_NCHK, 128), jnp.float32),
        pltpu.VMEM((_NCHK, 128), jnp.float32),
        pltpu.SemaphoreType.DMA,
        pltpu.SemaphoreType.DMA,
    ],
)
def _sc_gather(aa_hbm, yy_hbm, dist_hbm, slot_hbm, x_out,
               av, yv, keys, dkeys, tvals, wv, dv, xv, sem_w, sem_d):
    cid = lax.axis_index("c")
    sid = lax.axis_index("s")
    wid = cid * _NS + sid
    _build_keys(aa_hbm, yy_hbm, av, yv, keys, dkeys, tvals, cid, sid)

    def fire_g(c, carry):
        pltpu.make_async_copy(slot_hbm.at[keys.at[c]], wv.at[c], sem_w).start()
        pltpu.make_async_copy(dist_hbm.at[dkeys.at[c]], dv.at[c], sem_d).start()
        return carry

    lax.fori_loop(0, _NCHK, fire_g, 0)

    def drain_g(c, carry):
        pltpu.make_async_copy(slot_hbm.at[keys.at[c]], wv.at[c], sem_w).wait()
        pltpu.make_async_copy(dist_hbm.at[dkeys.at[c]], dv.at[c], sem_d).wait()
        return carry

    lax.fori_loop(0, _NCHK, drain_g, 0)

    sgn = (1 - 2 * cid).astype(jnp.float32)

    def comp(c, carry):
        for j in range(8):
            o = j * 16
            w = wv[c, pl.ds(o, 16)]
            t = tvals[c, pl.ds(o, 16)]
            d = dv[c, pl.ds(o, 16)]
            rep = (w == t) & (t < _P)
            xv[c, pl.ds(o, 16)] = jnp.where(rep, sgn * (0.5 - d),
                                            jnp.float32(-1e30))
        return carry

    lax.fori_loop(0, _NCHK, comp, 0)

    pltpu.sync_copy(xv, x_out.at[wid])


_ROWS = 256
_GRID = _B // _ROWS


def _dense_body(dist_ref, temb_ref, tmask_ref, out_ref):
    i = pl.program_id(0)

    @pl.when(i == 0)
    def _():
        out_ref[0, 0] = 0.0

    d = dist_ref[...]
    t = temb_ref[...]
    m = tmask_ref[...]
    dm = jnp.where(m, d, 0.0)
    sq = (dm - t) * (dm - t)
    rs = jnp.sum(sq, axis=1)
    anyk = jnp.any(m, axis=1)
    out_ref[0, 0] += jnp.sum(jnp.where(anyk, rs, 0.0))


_dense = pl.pallas_call(
    _dense_body,
    grid=(_GRID,),
    in_specs=[
        pl.BlockSpec((_ROWS, _B), lambda i: (i, 0)),
        pl.BlockSpec((_ROWS, _B), lambda i: (i, 0)),
        pl.BlockSpec((_ROWS, _B), lambda i: (i, 0)),
    ],
    out_specs=pl.BlockSpec(memory_space=pltpu.SMEM),
    out_shape=jax.ShapeDtypeStruct((1, 1), jnp.float32),
)


def _final_body(x_ref, s_ref, out_ref):
    x = x_ref[...]
    sp = jnp.log(jnp.exp(x) + 1.0)
    out_ref[0, 0] = (s_ref[0, 0] + jnp.sum(sp)) / _B


_final = pl.pallas_call(
    _final_body,
    in_specs=[
        pl.BlockSpec(memory_space=pltpu.VMEM),
        pl.BlockSpec(memory_space=pltpu.SMEM),
    ],
    out_specs=pl.BlockSpec(memory_space=pltpu.SMEM),
    out_shape=jax.ShapeDtypeStruct((1, 1), jnp.float32),
)


def _sc_pairs(aa, yy, dist_flat):
    (slot,) = _sc_scatter(aa, yy)
    (xvals,) = _sc_gather(aa, yy, dist_flat, slot)
    return xvals


def kernel(dist_mat, tree_embeds, tree_mask, a1, p, a2, n):
    pad = _PPAD - _P
    a1 = a1.astype(jnp.int32)
    p = p.astype(jnp.int32)
    a2 = a2.astype(jnp.int32)
    n = n.astype(jnp.int32)
    aa = jnp.concatenate([jnp.pad(a1, (0, pad)), jnp.pad(a2, (0, pad))])
    yy = jnp.concatenate([jnp.pad(p, (0, pad)), jnp.pad(n, (0, pad))])
    dist_flat = dist_mat.reshape(_B * _B)

    xvals = _sc_pairs(aa, yy, dist_flat)
    dsum = _dense(dist_mat, tree_embeds, tree_mask)
    total = _final(xvals.reshape(_NC * _NS * _NCHK, 128), dsum)
    return total[0, 0]


# speedup vs baseline: 3.6095x; 1.0941x over previous
import functools

import jax
import jax.numpy as jnp
from jax import lax
from jax.experimental import pallas as pl
from jax.experimental.pallas import tpu as pltpu
from jax.experimental.pallas import tpu_sc as plsc

_B = 4096
_P = 100000
_NC = 2
_NS = 16
_CH = 6272
_PPAD = _NS * _CH
_NCHK = _CH // 128
_SPARE = 4096
_SLOT_H = _B * _B + _SPARE

_mesh = plsc.VectorSubcoreMesh(core_axis_name="c", subcore_axis_name="s")


def _build_keys(aa_hbm, yy_hbm, av, yv, keys, dkeys, tvals, cid, sid):
    base = cid * _PPAD + sid * _CH
    tbase = sid * _CH
    slot_off = cid * _SLOT_H

    pltpu.sync_copy(aa_hbm.at[pl.ds(base, _CH)], av)
    pltpu.sync_copy(yy_hbm.at[pl.ds(base, _CH)], yv)

    def build(c, carry):
        for j in range(8):
            o = c * 128 + j * 16
            a = av[pl.ds(o, 16)]
            y = yv[pl.ds(o, 16)]
            t = tbase + o + lax.iota(jnp.int32, 16)
            valid = t < _P
            k = a * _B + y
            spare = _B * _B + (t & (_SPARE - 1))
            keys[c, pl.ds(j * 16, 16)] = jnp.where(valid, k, spare) + slot_off
            if dkeys is not None:
                dkeys[c, pl.ds(j * 16, 16)] = jnp.where(valid, k,
                                                        (t * 64) & (_B * _B - 1))
            tvals[c, pl.ds(j * 16, 16)] = t
        return carry

    lax.fori_loop(0, _NCHK, build, 0)


@functools.partial(
    pl.kernel,
    mesh=_mesh,
    out_type=[
        jax.ShapeDtypeStruct((_NC * _SLOT_H,), jnp.int32),
    ],
    scratch_types=[
        pltpu.VMEM((_CH,), jnp.int32),
        pltpu.VMEM((_CH,), jnp.int32),
        pltpu.VMEM((_NCHK, 128), jnp.int32),
        pltpu.VMEM((_NCHK, 128), jnp.int32),
        pltpu.SemaphoreType.DMA,
    ],
)
def _sc_scatter(aa_hbm, yy_hbm, slot_out, av, yv, keys, tvals, sem_s):
    cid = lax.axis_index("c")
    sid = lax.axis_index("s")
    _build_keys(aa_hbm, yy_hbm, av, yv, keys, None, tvals, cid, sid)

    def fire_s(c, carry):
        pltpu.make_async_copy(tvals.at[c], slot_out.at[keys.at[c]], sem_s).start()
        return carry

    lax.fori_loop(0, _NCHK, fire_s, 0)

    def drain_s(c, carry):
        pltpu.make_async_copy(tvals.at[c], slot_out.at[keys.at[c]], sem_s).wait()
        return carry

    lax.fori_loop(0, _NCHK, drain_s, 0)


@functools.partial(
    pl.kernel,
    mesh=_mesh,
    out_type=[
        jax.ShapeDtypeStruct((_NC * _NS, _NCHK, 128), jnp.float32),
    ],
    scratch_types=[
        pltpu.VMEM((_CH,), jnp.int32),
        pltpu.VMEM((_CH,), jnp.int32),
        pltpu.VMEM((_NCHK, 128), jnp.int32),
        pltpu.VMEM((_NCHK, 128), jnp.int32),
        pltpu.VMEM((_NCHK, 128), jnp.int32),
        pltpu.VMEM((_NCHK, 128), jnp.int32),
        pltpu.VMEM((_NCHK, 128), jnp.float32),
        pltpu.VMEM((_NCHK, 128), jnp.float32),
        pltpu.SemaphoreType.DMA,
        pltpu.SemaphoreType.DMA,
    ],
)
def _sc_gather(aa_hbm, yy_hbm, dist_hbm, slot_hbm, x_out,
               av, yv, keys, dkeys, tvals, wv, dv, xv, sem_w, sem_d):
    cid = lax.axis_index("c")
    sid = lax.axis_index("s")
    wid = cid * _NS + sid
    _build_keys(aa_hbm, yy_hbm, av, yv, keys, dkeys, tvals, cid, sid)

    def fire_g(c, carry):
        pltpu.make_async_copy(slot_hbm.at[keys.at[c]], wv.at[c], sem_w).start()
        pltpu.make_async_copy(dist_hbm.at[dkeys.at[c]], dv.at[c], sem_d).start()
        return carry

    lax.fori_loop(0, _NCHK, fire_g, 0)

    def drain_g(c, carry):
        pltpu.make_async_copy(slot_hbm.at[keys.at[c]], wv.at[c], sem_w).wait()
        pltpu.make_async_copy(dist_hbm.at[dkeys.at[c]], dv.at[c], sem_d).wait()
        return carry

    lax.fori_loop(0, _NCHK, drain_g, 0)

    sgn = (1 - 2 * cid).astype(jnp.float32)

    def comp(c, carry):
        for j in range(8):
            o = j * 16
            w = wv[c, pl.ds(o, 16)]
            t = tvals[c, pl.ds(o, 16)]
            d = dv[c, pl.ds(o, 16)]
            rep = (w == t) & (t < _P)
            xv[c, pl.ds(o, 16)] = jnp.where(rep, sgn * (0.5 - d),
                                            jnp.float32(-1e30))
        return carry

    lax.fori_loop(0, _NCHK, comp, 0)

    pltpu.sync_copy(xv, x_out.at[wid])


_ROWS = 256
_GRID = _B // _ROWS


def _dense_body(dist_ref, temb_ref, tmask_ref, out_ref):
    i = pl.program_id(0)

    @pl.when(i == 0)
    def _():
        out_ref[0, 0] = 0.0

    d = dist_ref[...]
    t = temb_ref[...]
    m = tmask_ref[...]
    dm = jnp.where(m, d, 0.0)
    sq = (dm - t) * (dm - t)
    rs = jnp.sum(sq, axis=1)
    anyk = jnp.any(m, axis=1)
    out_ref[0, 0] += jnp.sum(jnp.where(anyk, rs, 0.0))


_dense = pl.pallas_call(
    _dense_body,
    grid=(_GRID,),
    in_specs=[
        pl.BlockSpec((_ROWS, _B), lambda i: (i, 0)),
        pl.BlockSpec((_ROWS, _B), lambda i: (i, 0)),
        pl.BlockSpec((_ROWS, _B), lambda i: (i, 0)),
    ],
    out_specs=pl.BlockSpec(memory_space=pltpu.SMEM),
    out_shape=jax.ShapeDtypeStruct((1, 1), jnp.float32),
)


def _final_body(x_ref, s_ref, out_ref):
    x = x_ref[...]
    sp = jnp.log(jnp.exp(x) + 1.0)
    out_ref[0, 0] = (s_ref[0, 0] + jnp.sum(sp)) / _B


_final = pl.pallas_call(
    _final_body,
    in_specs=[
        pl.BlockSpec(memory_space=pltpu.VMEM),
        pl.BlockSpec(memory_space=pltpu.SMEM),
    ],
    out_specs=pl.BlockSpec(memory_space=pltpu.SMEM),
    out_shape=jax.ShapeDtypeStruct((1, 1), jnp.float32),
)


def _sc_pairs(aa, yy, dist_flat):
    (slot,) = _sc_scatter(aa, yy)
    (xvals,) = _sc_gather(aa, yy, dist_flat, slot)
    return xvals


def kernel(dist_mat, tree_embeds, tree_mask, a1, p, a2, n):
    pad = _PPAD - _P
    a1 = a1.astype(jnp.int32)
    p = p.astype(jnp.int32)
    a2 = a2.astype(jnp.int32)
    n = n.astype(jnp.int32)
    aa = jnp.concatenate([jnp.pad(a1, (0, pad)), jnp.pad(a2, (0, pad))])
    yy = jnp.concatenate([jnp.pad(p, (0, pad)), jnp.pad(n, (0, pad))])
    dist_flat = dist_mat.reshape(_B * _B)

    xvals = _sc_pairs(aa, yy, dist_flat)
    dsum = _dense(dist_mat, tree_embeds, tree_mask)
    total = _final(xvals.reshape(_NC * _NS * _NCHK, 128), dsum)
    return total[0, 0]


# speedup vs baseline: 6.5562x; 1.8164x over previous
import functools

import jax
import jax.numpy as jnp
from jax import lax
from jax.experimental import pallas as pl
from jax.experimental.pallas import tpu as pltpu
from jax.experimental.pallas import tpu_sc as plsc

_B = 4096
_P = 100000
_NC = 2
_NS = 16
_CH = 6272
_PPAD = _NS * _CH
_NCHK = _CH // 128
_SPARE = 4096
_SLOT_H = _B * _B + _SPARE
_NRING = 4

_mesh = plsc.VectorSubcoreMesh(core_axis_name="c", subcore_axis_name="s")


def _build_keys(aa_hbm, yy_hbm, av, yv, keys, dkeys, tvals, cid, sid):
    base = cid * _PPAD + sid * _CH
    tbase = sid * _CH

    pltpu.sync_copy(aa_hbm.at[pl.ds(base, _CH)], av)
    pltpu.sync_copy(yy_hbm.at[pl.ds(base, _CH)], yv)

    def build(c, carry):
        for j in range(8):
            o = c * 128 + j * 16
            a = av[pl.ds(o, 16)]
            y = yv[pl.ds(o, 16)]
            t = tbase + o + lax.iota(jnp.int32, 16)
            valid = t < _P
            k = a * _B + y
            if dkeys is None:
                spare = _B * _B + (t & (_SPARE - 1))
                keys[c, pl.ds(j * 16, 16)] = jnp.where(valid, k, spare)
            else:
                keys[c, pl.ds(j * 16, 16)] = jnp.where(valid, k * 128, 0)
                dkeys[c, pl.ds(j * 16, 16)] = jnp.where(valid, k,
                                                        (t * 64) & (_B * _B - 1))
            if tvals is not None:
                tvals[c, pl.ds(j * 16, 16)] = t
        return carry

    lax.fori_loop(0, _NCHK, build, 0)


@functools.partial(
    pl.kernel,
    mesh=_mesh,
    out_type=[
        jax.ShapeDtypeStruct((_SLOT_H, 128), jnp.int32),
        jax.ShapeDtypeStruct((_SLOT_H, 128), jnp.int32),
    ],
    scratch_types=[
        pltpu.VMEM((_CH,), jnp.int32),
        pltpu.VMEM((_CH,), jnp.int32),
        pltpu.VMEM((_NCHK, 128), jnp.int32),
        pltpu.VMEM((_NRING * 128, 128), jnp.int32),
        pltpu.SemaphoreType.DMA,
        pltpu.SemaphoreType.DMA,
        pltpu.SemaphoreType.DMA,
        pltpu.SemaphoreType.DMA,
    ],
)
def _sc_scatter(aa_hbm, yy_hbm, slot0_out, slot1_out,
                av, yv, keys, tvring, sem0, sem1, sem2, sem3):
    cid = lax.axis_index("c")
    sid = lax.axis_index("s")
    tbase = sid * _CH
    _build_keys(aa_hbm, yy_hbm, av, yv, keys, None, None, cid, sid)

    sems = (sem0, sem1, sem2, sem3)

    def run_scatter(slot_out):
        for c in range(_NCHK):
            s = c % _NRING
            if c >= _NRING:
                pltpu.make_async_copy(
                    tvring.at[pl.ds(s * 128, 128)],
                    slot_out.at[keys.at[c - _NRING]], sems[s]).wait()

            t0 = tbase + c * 128

            def fill(r, carry):
                tvring[s * 128 + r, pl.ds(0, 16)] = jnp.full(
                    (16,), t0 + r, jnp.int32)
                return carry

            lax.fori_loop(0, 128, fill, 0)

            pltpu.make_async_copy(tvring.at[pl.ds(s * 128, 128)],
                                  slot_out.at[keys.at[c]], sems[s]).start()

        for c in range(_NCHK - _NRING, _NCHK):
            s = c % _NRING
            pltpu.make_async_copy(tvring.at[pl.ds(s * 128, 128)],
                                  slot_out.at[keys.at[c]], sems[s]).wait()

    @pl.when(cid == 0)
    def _():
        run_scatter(slot0_out)

    @pl.when(cid == 1)
    def _():
        run_scatter(slot1_out)


@functools.partial(
    pl.kernel,
    mesh=_mesh,
    out_type=[
        jax.ShapeDtypeStruct((_NC * _NS, _NCHK, 128), jnp.float32),
    ],
    scratch_types=[
        pltpu.VMEM((_CH,), jnp.int32),
        pltpu.VMEM((_CH,), jnp.int32),
        pltpu.VMEM((_NCHK, 128), jnp.int32),
        pltpu.VMEM((_NCHK, 128), jnp.int32),
        pltpu.VMEM((_NCHK, 128), jnp.int32),
        pltpu.VMEM((_NCHK, 128), jnp.int32),
        pltpu.VMEM((_NCHK, 128), jnp.float32),
        pltpu.VMEM((_NCHK, 128), jnp.float32),
        pltpu.SemaphoreType.DMA,
        pltpu.SemaphoreType.DMA,
    ],
)
def _sc_gather(aa_hbm, yy_hbm, dist_hbm, slot0_hbm, slot1_hbm, x_out,
               av, yv, keys, dkeys, tvals, wv, dv, xv, sem_w, sem_d):
    cid = lax.axis_index("c")
    sid = lax.axis_index("s")
    wid = cid * _NS + sid
    _build_keys(aa_hbm, yy_hbm, av, yv, keys, dkeys, tvals, cid, sid)

    def fire_d(c, carry):
        pltpu.make_async_copy(dist_hbm.at[dkeys.at[c]], dv.at[c], sem_d).start()
        return carry

    lax.fori_loop(0, _NCHK, fire_d, 0)

    def run_gather(slot_hbm):
        def fire_w(c, carry):
            pltpu.make_async_copy(slot_hbm.at[keys.at[c]], wv.at[c],
                                  sem_w).start()
            return carry

        lax.fori_loop(0, _NCHK, fire_w, 0)

        def drain_w(c, carry):
            pltpu.make_async_copy(slot_hbm.at[keys.at[c]], wv.at[c],
                                  sem_w).wait()
            return carry

        lax.fori_loop(0, _NCHK, drain_w, 0)

    @pl.when(cid == 0)
    def _():
        run_gather(slot0_hbm)

    @pl.when(cid == 1)
    def _():
        run_gather(slot1_hbm)

    def drain_d(c, carry):
        pltpu.make_async_copy(dist_hbm.at[dkeys.at[c]], dv.at[c], sem_d).wait()
        return carry

    lax.fori_loop(0, _NCHK, drain_d, 0)

    sgn = (1 - 2 * cid).astype(jnp.float32)

    def comp(c, carry):
        for j in range(8):
            o = j * 16
            w = wv[c, pl.ds(o, 16)]
            t = tvals[c, pl.ds(o, 16)]
            d = dv[c, pl.ds(o, 16)]
            rep = (w == t) & (t < _P)
            xv[c, pl.ds(o, 16)] = jnp.where(rep, sgn * (0.5 - d),
                                            jnp.float32(-1e30))
        return carry

    lax.fori_loop(0, _NCHK, comp, 0)

    pltpu.sync_copy(xv, x_out.at[wid])


_ROWS = 256
_GRID = _B // _ROWS


def _dense_body(dist_ref, temb_ref, tmask_ref, out_ref):
    i = pl.program_id(0)

    @pl.when(i == 0)
    def _():
        out_ref[0, 0] = 0.0

    d = dist_ref[...]
    t = temb_ref[...]
    m = tmask_ref[...]
    dm = jnp.where(m, d, 0.0)
    sq = (dm - t) * (dm - t)
    rs = jnp.sum(sq, axis=1)
    anyk = jnp.any(m, axis=1)
    out_ref[0, 0] += jnp.sum(jnp.where(anyk, rs, 0.0))


_dense = pl.pallas_call(
    _dense_body,
    grid=(_GRID,),
    in_specs=[
        pl.BlockSpec((_ROWS, _B), lambda i: (i, 0)),
        pl.BlockSpec((_ROWS, _B), lambda i: (i, 0)),
        pl.BlockSpec((_ROWS, _B), lambda i: (i, 0)),
    ],
    out_specs=pl.BlockSpec(memory_space=pltpu.SMEM),
    out_shape=jax.ShapeDtypeStruct((1, 1), jnp.float32),
)


def _final_body(x_ref, s_ref, out_ref):
    x = x_ref[...]
    sp = jnp.log(jnp.exp(x) + 1.0)
    out_ref[0, 0] = (s_ref[0, 0] + jnp.sum(sp)) / _B


_final = pl.pallas_call(
    _final_body,
    in_specs=[
        pl.BlockSpec(memory_space=pltpu.VMEM),
        pl.BlockSpec(memory_space=pltpu.SMEM),
    ],
    out_specs=pl.BlockSpec(memory_space=pltpu.SMEM),
    out_shape=jax.ShapeDtypeStruct((1, 1), jnp.float32),
)


def _sc_pairs(aa, yy, dist_flat):
    slot0, slot1 = _sc_scatter(aa, yy)
    (xvals,) = _sc_gather(aa, yy, dist_flat,
                          slot0.reshape(_SLOT_H * 128),
                          slot1.reshape(_SLOT_H * 128))
    return xvals


def kernel(dist_mat, tree_embeds, tree_mask, a1, p, a2, n):
    pad = _PPAD - _P
    a1 = a1.astype(jnp.int32)
    p = p.astype(jnp.int32)
    a2 = a2.astype(jnp.int32)
    n = n.astype(jnp.int32)
    aa = jnp.concatenate([jnp.pad(a1, (0, pad)), jnp.pad(a2, (0, pad))])
    yy = jnp.concatenate([jnp.pad(p, (0, pad)), jnp.pad(n, (0, pad))])
    dist_flat = dist_mat.reshape(_B * _B)

    xvals = _sc_pairs(aa, yy, dist_flat)
    dsum = _dense(dist_mat, tree_embeds, tree_mask)
    total = _final(xvals.reshape(_NC * _NS * _NCHK, 128), dsum)
    return total[0, 0]


# speedup vs baseline: 7.4850x; 1.1417x over previous
import functools

import jax
import jax.numpy as jnp
from jax import lax
from jax.experimental import pallas as pl
from jax.experimental.pallas import tpu as pltpu
from jax.experimental.pallas import tpu_sc as plsc

_B = 4096
_P = 100000
_NC = 2
_NS = 16
_CH = 6272
_PPAD = _NS * _CH
_NCHK = _CH // 128
_SPARE = 4096
_SLOT_H = _B * _B + _SPARE
_NRING = 4

_mesh = plsc.VectorSubcoreMesh(core_axis_name="c", subcore_axis_name="s")


def _build_keys(aa_hbm, yy_hbm, av, yv, keys, dkeys, tvals, cid, sid):
    base = cid * _PPAD + sid * _CH
    tbase = sid * _CH

    pltpu.sync_copy(aa_hbm.at[pl.ds(base, _CH)], av)
    pltpu.sync_copy(yy_hbm.at[pl.ds(base, _CH)], yv)

    def build(c, carry):
        for j in range(8):
            o = c * 128 + j * 16
            a = av[pl.ds(o, 16)]
            y = yv[pl.ds(o, 16)]
            t = tbase + o + lax.iota(jnp.int32, 16)
            valid = t < _P
            k = a * _B + y
            if dkeys is None:
                spare = _B * _B + (t & (_SPARE - 1))
                keys[c, pl.ds(j * 16, 16)] = jnp.where(valid, k, spare)
            else:
                keys[c, pl.ds(j * 16, 16)] = jnp.where(valid, k * 128, 0)
                dkeys[c, pl.ds(j * 16, 16)] = jnp.where(valid, k,
                                                        (t * 64) & (_B * _B - 1))
            if tvals is not None:
                tvals[c, pl.ds(j * 16, 16)] = t
        return carry

    lax.fori_loop(0, _NCHK, build, 0)


@functools.partial(
    pl.kernel,
    mesh=_mesh,
    out_type=[
        jax.ShapeDtypeStruct((_SLOT_H, 128), jnp.int32),
        jax.ShapeDtypeStruct((_SLOT_H, 128), jnp.int32),
    ],
    scratch_types=[
        pltpu.VMEM((_CH,), jnp.int32),
        pltpu.VMEM((_CH,), jnp.int32),
        pltpu.VMEM((_NCHK, 128), jnp.int32),
        pltpu.VMEM((_NRING * 128, 128), jnp.int32),
        pltpu.SemaphoreType.DMA,
        pltpu.SemaphoreType.DMA,
        pltpu.SemaphoreType.DMA,
        pltpu.SemaphoreType.DMA,
    ],
)
def _sc_scatter(aa_hbm, yy_hbm, slot0_out, slot1_out,
                av, yv, keys, tvring, sem0, sem1, sem2, sem3):
    cid = lax.axis_index("c")
    sid = lax.axis_index("s")
    tbase = sid * _CH
    _build_keys(aa_hbm, yy_hbm, av, yv, keys, None, None, cid, sid)

    sems = (sem0, sem1, sem2, sem3)

    def run_scatter(slot_out):
        for c in range(_NCHK):
            s = c % _NRING
            if c >= _NRING:
                pltpu.make_async_copy(
                    tvring.at[pl.ds(s * 128, 128)],
                    slot_out.at[keys.at[c - _NRING]], sems[s]).wait()

            t0 = tbase + c * 128

            def fill(r, carry):
                tvring[s * 128 + r, pl.ds(0, 16)] = jnp.full(
                    (16,), t0 + r, jnp.int32)
                return carry

            lax.fori_loop(0, 128, fill, 0)

            pltpu.make_async_copy(tvring.at[pl.ds(s * 128, 128)],
                                  slot_out.at[keys.at[c]], sems[s]).start()

        for c in range(_NCHK - _NRING, _NCHK):
            s = c % _NRING
            pltpu.make_async_copy(tvring.at[pl.ds(s * 128, 128)],
                                  slot_out.at[keys.at[c]], sems[s]).wait()

    @pl.when(cid == 0)
    def _():
        run_scatter(slot0_out)

    @pl.when(cid == 1)
    def _():
        run_scatter(slot1_out)


@functools.partial(
    pl.kernel,
    mesh=_mesh,
    out_type=[
        jax.ShapeDtypeStruct((_NC * _NS, _NCHK, 128), jnp.float32),
    ],
    scratch_types=[
        pltpu.VMEM((_CH,), jnp.int32),
        pltpu.VMEM((_CH,), jnp.int32),
        pltpu.VMEM((_NCHK, 128), jnp.int32),
        pltpu.VMEM((_NCHK, 128), jnp.int32),
        pltpu.VMEM((_NCHK, 128), jnp.int32),
        pltpu.VMEM((_NCHK, 128), jnp.int32),
        pltpu.VMEM((_NCHK, 128), jnp.float32),
        pltpu.VMEM((_NCHK, 128), jnp.float32),
        pltpu.SemaphoreType.DMA,
        pltpu.SemaphoreType.DMA,
    ],
)
def _sc_gather(aa_hbm, yy_hbm, dist_hbm, slot0_hbm, slot1_hbm, x_out,
               av, yv, keys, dkeys, tvals, wv, dv, xv, sem_w, sem_d):
    cid = lax.axis_index("c")
    sid = lax.axis_index("s")
    wid = cid * _NS + sid
    _build_keys(aa_hbm, yy_hbm, av, yv, keys, dkeys, tvals, cid, sid)

    def fire_d(c, carry):
        pltpu.make_async_copy(dist_hbm.at[dkeys.at[c]], dv.at[c], sem_d).start()
        return carry

    lax.fori_loop(0, _NCHK, fire_d, 0)

    def run_gather(slot_hbm):
        def fire_w(c, carry):
            pltpu.make_async_copy(slot_hbm.at[keys.at[c]], wv.at[c],
                                  sem_w).start()
            return carry

        lax.fori_loop(0, _NCHK, fire_w, 0)

        def drain_w(c, carry):
            pltpu.make_async_copy(slot_hbm.at[keys.at[c]], wv.at[c],
                                  sem_w).wait()
            return carry

        lax.fori_loop(0, _NCHK, drain_w, 0)

    @pl.when(cid == 0)
    def _():
        run_gather(slot0_hbm)

    @pl.when(cid == 1)
    def _():
        run_gather(slot1_hbm)

    def drain_d(c, carry):
        pltpu.make_async_copy(dist_hbm.at[dkeys.at[c]], dv.at[c], sem_d).wait()
        return carry

    lax.fori_loop(0, _NCHK, drain_d, 0)

    sgn = (1 - 2 * cid).astype(jnp.float32)

    def comp(c, carry):
        for j in range(8):
            o = j * 16
            w = wv[c, pl.ds(o, 16)]
            t = tvals[c, pl.ds(o, 16)]
            d = dv[c, pl.ds(o, 16)]
            rep = (w == t) & (t < _P)
            xv[c, pl.ds(o, 16)] = jnp.where(rep, sgn * (0.5 - d),
                                            jnp.float32(-1e30))
        return carry

    lax.fori_loop(0, _NCHK, comp, 0)

    pltpu.sync_copy(xv, x_out.at[wid])


_ROWS = 256
_GRID = _B // _ROWS


def _dense_body(dist_ref, temb_ref, tmask_ref, out_ref, lin_ref):
    i = pl.program_id(0)

    @pl.when(i == 0)
    def _():
        out_ref[0, 0] = 0.0

    d = dist_ref[...]
    t = temb_ref[...]
    m = tmask_ref[...]
    dm = jnp.where(m, d, 0.0)
    sq = (dm - t) * (dm - t)
    rs = jnp.sum(sq, axis=1)
    anyk = jnp.any(m, axis=1)
    out_ref[0, 0] += jnp.sum(jnp.where(anyk, rs, 0.0))
    lin_ref[...] = d.reshape(_ROWS * 32, 128)


_dense = pl.pallas_call(
    _dense_body,
    grid=(_GRID,),
    in_specs=[
        pl.BlockSpec((_ROWS, _B), lambda i: (i, 0)),
        pl.BlockSpec((_ROWS, _B), lambda i: (i, 0)),
        pl.BlockSpec((_ROWS, _B), lambda i: (i, 0)),
    ],
    out_specs=[
        pl.BlockSpec(memory_space=pltpu.SMEM),
        pl.BlockSpec((_ROWS * 32, 128), lambda i: (i, 0)),
    ],
    out_shape=[
        jax.ShapeDtypeStruct((1, 1), jnp.float32),
        jax.ShapeDtypeStruct((_B * _B // 128, 128), jnp.float32),
    ],
)


def _final_body(x_ref, s_ref, out_ref):
    x = x_ref[...]
    sp = jnp.log(jnp.exp(x) + 1.0)
    out_ref[0, 0] = (s_ref[0, 0] + jnp.sum(sp)) / _B


_final = pl.pallas_call(
    _final_body,
    in_specs=[
        pl.BlockSpec(memory_space=pltpu.VMEM),
        pl.BlockSpec(memory_space=pltpu.SMEM),
    ],
    out_specs=pl.BlockSpec(memory_space=pltpu.SMEM),
    out_shape=jax.ShapeDtypeStruct((1, 1), jnp.float32),
)


def _sc_pairs(aa, yy, dist_flat):
    slot0, slot1 = _sc_scatter(aa, yy)
    (xvals,) = _sc_gather(aa, yy, dist_flat,
                          slot0.reshape(_SLOT_H * 128),
                          slot1.reshape(_SLOT_H * 128))
    return xvals


def kernel(dist_mat, tree_embeds, tree_mask, a1, p, a2, n):
    pad = _PPAD - _P
    a1 = a1.astype(jnp.int32)
    p = p.astype(jnp.int32)
    a2 = a2.astype(jnp.int32)
    n = n.astype(jnp.int32)
    aa = jnp.concatenate([jnp.pad(a1, (0, pad)), jnp.pad(a2, (0, pad))])
    yy = jnp.concatenate([jnp.pad(p, (0, pad)), jnp.pad(n, (0, pad))])

    dsum, dist_lin = _dense(dist_mat, tree_embeds, tree_mask)
    xvals = _sc_pairs(aa, yy, dist_lin.reshape(_B * _B))
    total = _final(xvals.reshape(_NC * _NS * _NCHK, 128), dsum)
    return total[0, 0]
